# baseline (speedup 1.0000x reference)
import jax
import jax.numpy as jnp
from jax import lax
from jax.experimental import pallas as pl
from jax.experimental.pallas import tpu as pltpu

_N_TOP = 5
_ROWS_PER_BLOCK = 8
_STRIP = 256


def _full_scan_tops(x_ref, lane, neg, big):
    r, t = x_ref.shape
    nchunk = t // 128
    nset = 4
    unroll = min(32, nchunk)

    def body(i, carry):
        m, a = carry
        m = [list(s) for s in m]
        a = [list(s) for s in a]
        base = i * (unroll * 128)
        for u in range(unroll):
            s = u % nset
            off = pl.multiple_of(base + u * 128, 128)
            tv = x_ref[:, pl.ds(off, 128)]
            ta = lane + off
            cc = [tv > m[s][j] for j in range(_N_TOP)]
            nm = [jnp.where(cc[0], tv, m[s][0])]
            na = [jnp.where(cc[0], ta, a[s][0])]
            for j in range(1, _N_TOP):
                nm.append(jnp.where(cc[j],
                                    jnp.where(cc[j - 1], m[s][j - 1], tv),
                                    m[s][j]))
                na.append(jnp.where(cc[j],
                                    jnp.where(cc[j - 1], a[s][j - 1], ta),
                                    a[s][j]))
            m[s], a[s] = nm, na
        return (tuple(tuple(s) for s in m), tuple(tuple(s) for s in a))

    m0 = tuple(tuple(neg for _ in range(_N_TOP)) for _ in range(nset))
    a0 = tuple(tuple(big for _ in range(_N_TOP)) for _ in range(nset))
    m, a = lax.fori_loop(0, nchunk // unroll, body, (m0, a0))
    m = [list(s) for s in m]
    a = [list(s) for s in a]
    mm, aa = m[0], a[0]
    for s in range(1, nset):
        for j2 in range(_N_TOP):
            tv, ta = m[s][j2], a[s][j2]
            for j in range(_N_TOP):
                swap = (tv > mm[j]) | ((tv == mm[j]) & (ta < aa[j]))
                mm[j], tv = (jnp.where(swap, tv, mm[j]),
                             jnp.where(swap, mm[j], tv))
                aa[j], ta = (jnp.where(swap, ta, aa[j]),
                             jnp.where(swap, aa[j], ta))
    tops = jnp.zeros((r, 128), jnp.int32)
    for k in range(_N_TOP):
        bv = jnp.max(mm[0], axis=1, keepdims=True)
        elig = mm[0] == bv
        bcol = jnp.min(jnp.where(elig, aa[0], t), axis=1, keepdims=True)
        tops = jnp.where(lane == k, bcol, tops)
        pop = elig & (aa[0] == bcol)
        for j in range(_N_TOP - 1):
            mm[j] = jnp.where(pop, mm[j + 1], mm[j])
            aa[j] = jnp.where(pop, aa[j + 1], aa[j])
        mm[_N_TOP - 1] = jnp.where(pop, neg, mm[_N_TOP - 1])
        aa[_N_TOP - 1] = jnp.where(pop, big, aa[_N_TOP - 1])
    return tops


def _tops_kernel(x_ref, t_ref):
    r, t = x_ref.shape
    nchunk = t // 128
    nset = 4
    lane = lax.broadcasted_iota(jnp.int32, (r, 128), 1)
    neg = jnp.full((r, 128), -jnp.inf, jnp.float32)
    big = jnp.full((r, 128), t, jnp.int32)
    unroll = min(32, nchunk)

    def body(i, carry):
        m1, a1, m2, a2, m3 = (list(v) for v in carry)
        base = i * (unroll * 128)
        for u in range(unroll):
            s = u % nset
            off = pl.multiple_of(base + u * 128, 128)
            tv = x_ref[:, pl.ds(off, 128)]
            ta = lane + off
            c1 = tv > m1[s]
            c2 = tv > m2[s]
            c3 = tv > m3[s]
            m3[s] = jnp.where(c3, jnp.where(c2, m2[s], tv), m3[s])
            m2[s], a2[s] = (jnp.where(c2, jnp.where(c1, m1[s], tv), m2[s]),
                            jnp.where(c2, jnp.where(c1, a1[s], ta), a2[s]))
            m1[s], a1[s] = (jnp.where(c1, tv, m1[s]),
                            jnp.where(c1, ta, a1[s]))
        return tuple(tuple(v) for v in (m1, a1, m2, a2, m3))

    init = (tuple(neg for _ in range(nset)), tuple(big for _ in range(nset)),
            tuple(neg for _ in range(nset)), tuple(big for _ in range(nset)),
            tuple(neg for _ in range(nset)))
    m1, a1, m2, a2, m3 = (list(v) for v in
                          lax.fori_loop(0, nchunk // unroll, body, init))

    def lexgt(v1, c1_, v2, c2_):
        return (v1 > v2) | ((v1 == v2) & (c1_ < c2_))

    def merge(x, y):
        xm1, xa1, xm2, xa2, xm3 = x
        ym1, ya1, ym2, ya2, ym3 = y
        g1 = lexgt(ym1, ya1, xm1, xa1)
        h1, ha1 = jnp.where(g1, ym1, xm1), jnp.where(g1, ya1, xa1)
        w2, wa2 = jnp.where(g1, ym2, xm2), jnp.where(g1, ya2, xa2)
        w3 = jnp.where(g1, ym3, xm3)
        l1, la1 = jnp.where(g1, xm1, ym1), jnp.where(g1, xa1, ya1)
        l2, la2 = jnp.where(g1, xm2, ym2), jnp.where(g1, xa2, ya2)
        l3 = jnp.where(g1, xm3, ym3)
        g2 = lexgt(w2, wa2, l1, la1)
        h2 = jnp.where(g2, w2, l1)
        ha2 = jnp.where(g2, wa2, la1)
        del l3
        h3 = jnp.where(g2, jnp.maximum(w3, l1), jnp.maximum(w2, l2))
        return (h1, ha1, h2, ha2, h3)

    acc = (m1[0], a1[0], m2[0], a2[0], m3[0])
    for s in range(1, nset):
        acc = merge(acc, (m1[s], a1[s], m2[s], a2[s], m3[s]))
    hh, ah, nn, an, third = acc

    tops = jnp.zeros((r, 128), jnp.int32)
    bv = jnp.max(hh, axis=1, keepdims=True)
    for k in range(_N_TOP):
        elig = hh == bv
        bcol = jnp.min(jnp.where(elig, ah, t), axis=1, keepdims=True)
        tops = jnp.where(lane == k, bcol, tops)
        pop = elig & (ah == bcol)
        hh = jnp.where(pop, nn, hh)
        ah = jnp.where(pop, an, ah)
        nn = jnp.where(pop, third, nn)
        an = jnp.where(pop, big, an)
        if k < _N_TOP - 1:
            bv = jnp.max(hh, axis=1, keepdims=True)
    p5 = bv
    badrow = jnp.max(third, axis=1, keepdims=True) >= p5
    bad = jnp.max(jnp.where(badrow, 1, 0))

    @pl.when(bad == 0)
    def _():
        t_ref[:] = tops

    @pl.when(bad != 0)
    def _():
        t_ref[:] = _full_scan_tops(x_ref, lane, neg, big)


def _scatter_kernel(tops_ref, bw_ref, o_ref):
    gr = pl.program_id(0)
    r, t = o_ref.shape
    o_ref[:] = jnp.zeros((r, t), jnp.float32)
    std = bw_ref[0]
    inv = 1.0 / std
    scale = inv / jnp.sqrt(2.0 * jnp.pi)
    j = lax.broadcasted_iota(jnp.int32, (1, _STRIP), 1).astype(jnp.float32)
    for row in range(r):
        for k in range(_N_TOP):
            idx = tops_ref[gr * r + row, k]
            s = jnp.clip((idx - _STRIP // 8) // 128 * 128, 0, t - _STRIP)
            s = pl.multiple_of(s, 128)
            d = (j + s.astype(jnp.float32) - idx.astype(jnp.float32)) * inv
            vals = jnp.exp(-0.5 * d * d) * scale
            cur = o_ref[pl.ds(row, 1), pl.ds(s, _STRIP)]
            o_ref[pl.ds(row, 1), pl.ds(s, _STRIP)] = cur + vals


@jax.jit
def kernel(inputs, bw):
    b, t = inputs.shape
    grid = b // _ROWS_PER_BLOCK
    tops = pl.pallas_call(
        _tops_kernel,
        grid=(grid,),
        in_specs=[pl.BlockSpec((_ROWS_PER_BLOCK, t), lambda i: (i, 0))],
        out_specs=pl.BlockSpec((_ROWS_PER_BLOCK, 128), lambda i: (i, 0)),
        out_shape=jax.ShapeDtypeStruct((b, 128), jnp.int32),
    )(inputs)
    srows = 32
    return pl.pallas_call(
        _scatter_kernel,
        grid=(b // srows,),
        in_specs=[
            pl.BlockSpec(memory_space=pltpu.SMEM),
            pl.BlockSpec(memory_space=pltpu.SMEM),
        ],
        out_specs=pl.BlockSpec((srows, t), lambda i: (i, 0)),
        out_shape=jax.ShapeDtypeStruct((b, t), jnp.float32),
    )(tops, bw)


# speedup vs baseline: 1.0106x; 1.0106x over previous
import jax
import jax.numpy as jnp
from jax import lax
from jax.experimental import pallas as pl
from jax.experimental.pallas import tpu as pltpu

_N_TOP = 5
_ROWS_PER_BLOCK = 8
_STRIP = 256


def _full_scan_tops(x_ref, lane, neg, big):
    r, t = x_ref.shape
    nchunk = t // 128
    nset = 4
    unroll = min(32, nchunk)

    def body(i, carry):
        m, a = carry
        m = [list(s) for s in m]
        a = [list(s) for s in a]
        base = i * (unroll * 128)
        for u in range(unroll):
            s = u % nset
            off = pl.multiple_of(base + u * 128, 128)
            tv = x_ref[:, pl.ds(off, 128)]
            ta = lane + off
            cc = [tv > m[s][j] for j in range(_N_TOP)]
            nm = [jnp.where(cc[0], tv, m[s][0])]
            na = [jnp.where(cc[0], ta, a[s][0])]
            for j in range(1, _N_TOP):
                nm.append(jnp.where(cc[j],
                                    jnp.where(cc[j - 1], m[s][j - 1], tv),
                                    m[s][j]))
                na.append(jnp.where(cc[j],
                                    jnp.where(cc[j - 1], a[s][j - 1], ta),
                                    a[s][j]))
            m[s], a[s] = nm, na
        return (tuple(tuple(s) for s in m), tuple(tuple(s) for s in a))

    m0 = tuple(tuple(neg for _ in range(_N_TOP)) for _ in range(nset))
    a0 = tuple(tuple(big for _ in range(_N_TOP)) for _ in range(nset))
    m, a = lax.fori_loop(0, nchunk // unroll, body, (m0, a0))
    m = [list(s) for s in m]
    a = [list(s) for s in a]
    mm, aa = m[0], a[0]
    for s in range(1, nset):
        for j2 in range(_N_TOP):
            tv, ta = m[s][j2], a[s][j2]
            for j in range(_N_TOP):
                swap = (tv > mm[j]) | ((tv == mm[j]) & (ta < aa[j]))
                mm[j], tv = (jnp.where(swap, tv, mm[j]),
                             jnp.where(swap, mm[j], tv))
                aa[j], ta = (jnp.where(swap, ta, aa[j]),
                             jnp.where(swap, aa[j], ta))
    tops = jnp.zeros((r, 128), jnp.int32)
    for k in range(_N_TOP):
        bv = jnp.max(mm[0], axis=1, keepdims=True)
        elig = mm[0] == bv
        bcol = jnp.min(jnp.where(elig, aa[0], t), axis=1, keepdims=True)
        tops = jnp.where(lane == k, bcol, tops)
        pop = elig & (aa[0] == bcol)
        for j in range(_N_TOP - 1):
            mm[j] = jnp.where(pop, mm[j + 1], mm[j])
            aa[j] = jnp.where(pop, aa[j + 1], aa[j])
        mm[_N_TOP - 1] = jnp.where(pop, neg, mm[_N_TOP - 1])
        aa[_N_TOP - 1] = jnp.where(pop, big, aa[_N_TOP - 1])
    return tops


def _tops_kernel(x_ref, t_ref):
    r, t = x_ref.shape
    nchunk = t // 128
    nset = 4
    lane = lax.broadcasted_iota(jnp.int32, (r, 128), 1)
    neg = jnp.full((r, 128), -jnp.inf, jnp.float32)
    big = jnp.full((r, 128), t, jnp.int32)
    unroll = min(64, nchunk)

    def body(i, carry):
        m1, a1, m2, a2, m3 = (list(v) for v in carry)
        base = i * (unroll * 128)
        for u in range(unroll):
            s = u % nset
            off = pl.multiple_of(base + u * 128, 128)
            tv = x_ref[:, pl.ds(off, 128)]
            ta = lane + off
            c1 = tv > m1[s]
            c2 = tv > m2[s]
            c3 = tv > m3[s]
            m3[s] = jnp.where(c3, jnp.where(c2, m2[s], tv), m3[s])
            m2[s], a2[s] = (jnp.where(c2, jnp.where(c1, m1[s], tv), m2[s]),
                            jnp.where(c2, jnp.where(c1, a1[s], ta), a2[s]))
            m1[s], a1[s] = (jnp.where(c1, tv, m1[s]),
                            jnp.where(c1, ta, a1[s]))
        return tuple(tuple(v) for v in (m1, a1, m2, a2, m3))

    init = (tuple(neg for _ in range(nset)), tuple(big for _ in range(nset)),
            tuple(neg for _ in range(nset)), tuple(big for _ in range(nset)),
            tuple(neg for _ in range(nset)))
    m1, a1, m2, a2, m3 = (list(v) for v in
                          lax.fori_loop(0, nchunk // unroll, body, init))

    def lexgt(v1, c1_, v2, c2_):
        return (v1 > v2) | ((v1 == v2) & (c1_ < c2_))

    def merge(x, y):
        xm1, xa1, xm2, xa2, xm3 = x
        ym1, ya1, ym2, ya2, ym3 = y
        g1 = lexgt(ym1, ya1, xm1, xa1)
        h1, ha1 = jnp.where(g1, ym1, xm1), jnp.where(g1, ya1, xa1)
        w2, wa2 = jnp.where(g1, ym2, xm2), jnp.where(g1, ya2, xa2)
        w3 = jnp.where(g1, ym3, xm3)
        l1, la1 = jnp.where(g1, xm1, ym1), jnp.where(g1, xa1, ya1)
        l2, la2 = jnp.where(g1, xm2, ym2), jnp.where(g1, xa2, ya2)
        l3 = jnp.where(g1, xm3, ym3)
        g2 = lexgt(w2, wa2, l1, la1)
        h2 = jnp.where(g2, w2, l1)
        ha2 = jnp.where(g2, wa2, la1)
        del l3
        h3 = jnp.where(g2, jnp.maximum(w3, l1), jnp.maximum(w2, l2))
        return (h1, ha1, h2, ha2, h3)

    acc = (m1[0], a1[0], m2[0], a2[0], m3[0])
    for s in range(1, nset):
        acc = merge(acc, (m1[s], a1[s], m2[s], a2[s], m3[s]))
    hh, ah, nn, an, third = acc

    tops = jnp.zeros((r, 128), jnp.int32)
    bv = jnp.max(hh, axis=1, keepdims=True)
    for k in range(_N_TOP):
        elig = hh == bv
        bcol = jnp.min(jnp.where(elig, ah, t), axis=1, keepdims=True)
        tops = jnp.where(lane == k, bcol, tops)
        pop = elig & (ah == bcol)
        hh = jnp.where(pop, nn, hh)
        ah = jnp.where(pop, an, ah)
        nn = jnp.where(pop, third, nn)
        an = jnp.where(pop, big, an)
        if k < _N_TOP - 1:
            bv = jnp.max(hh, axis=1, keepdims=True)
    p5 = bv
    badrow = jnp.max(third, axis=1, keepdims=True) >= p5
    bad = jnp.max(jnp.where(badrow, 1, 0))

    @pl.when(bad == 0)
    def _():
        t_ref[:] = tops

    @pl.when(bad != 0)
    def _():
        t_ref[:] = _full_scan_tops(x_ref, lane, neg, big)


def _scatter_kernel(tops_ref, bw_ref, o_ref):
    gr = pl.program_id(0)
    r, t = o_ref.shape
    o_ref[:] = jnp.zeros((r, t), jnp.float32)
    std = bw_ref[0]
    inv = 1.0 / std
    scale = inv / jnp.sqrt(2.0 * jnp.pi)
    j = lax.broadcasted_iota(jnp.int32, (1, _STRIP), 1).astype(jnp.float32)
    for row in range(r):
        for k in range(_N_TOP):
            idx = tops_ref[gr * r + row, k]
            s = jnp.clip((idx - _STRIP // 8) // 128 * 128, 0, t - _STRIP)
            s = pl.multiple_of(s, 128)
            d = (j + s.astype(jnp.float32) - idx.astype(jnp.float32)) * inv
            vals = jnp.exp(-0.5 * d * d) * scale
            cur = o_ref[pl.ds(row, 1), pl.ds(s, _STRIP)]
            o_ref[pl.ds(row, 1), pl.ds(s, _STRIP)] = cur + vals


@jax.jit
def kernel(inputs, bw):
    b, t = inputs.shape
    grid = b // _ROWS_PER_BLOCK
    tops = pl.pallas_call(
        _tops_kernel,
        grid=(grid,),
        in_specs=[pl.BlockSpec((_ROWS_PER_BLOCK, t), lambda i: (i, 0))],
        out_specs=pl.BlockSpec((_ROWS_PER_BLOCK, 128), lambda i: (i, 0)),
        out_shape=jax.ShapeDtypeStruct((b, 128), jnp.int32),
    )(inputs)
    srows = 16
    return pl.pallas_call(
        _scatter_kernel,
        grid=(b // srows,),
        in_specs=[
            pl.BlockSpec(memory_space=pltpu.SMEM),
            pl.BlockSpec(memory_space=pltpu.SMEM),
        ],
        out_specs=pl.BlockSpec((srows, t), lambda i: (i, 0)),
        out_shape=jax.ShapeDtypeStruct((b, t), jnp.float32),
    )(tops, bw)
